# baseline (speedup 1.0000x reference)
import functools

import jax
import jax.numpy as jnp
from jax import lax
from jax.experimental import pallas as pl
from jax.experimental.pallas import tpu as pltpu

_K = 16
_ITERS = 3


def _kmedoids_body(x_ref, w_ref, out_ref, xt3_ref, d_ref):
    xb = x_ref[0]
    wb = w_ref[0]
    F, S = xb.shape
    k = _K
    G = F // 8
    CH = 64

    w_tok = jnp.mean(wb, axis=0, keepdims=True)

    for g in range(G):
        xt3_ref[g] = xb[8 * g : 8 * g + 8, :].T

    for ci in range(S // CH):
        def g_body(g, acc, ci=ci):
            xg = x_ref[0, pl.ds(g * 8, 8), :]
            xtc = xt3_ref[g, ci * CH : (ci + 1) * CH, :]
            for u in range(8):
                col = xtc[:, u : u + 1]
                row = xg[u : u + 1, :]
                acc = acc + jnp.abs(col - row)
            return acc

        acc = lax.fori_loop(0, G, g_body, jnp.zeros((CH, S), jnp.float32))
        d_ref[ci * CH : (ci + 1) * CH, :] = acc

    lane_iota = lax.broadcasted_iota(jnp.int32, (1, S), 1)
    k_iota = lax.broadcasted_iota(jnp.int32, (1, k), 1)

    def topk_body(c, carry):
        w_cur, ctr = carry
        m = jnp.max(w_cur)
        idx = jnp.min(jnp.where(w_cur == m, lane_iota, S))
        ctr = jnp.where(k_iota == c, idx, ctr)
        w_cur = jnp.where(lane_iota == idx, -jnp.inf, w_cur)
        return w_cur, ctr

    _, ctr = lax.fori_loop(
        0, k, topk_body, (w_tok, jnp.zeros((1, k), jnp.int32))
    )

    s_iota_col = lax.broadcasted_iota(jnp.int32, (S, 1), 0)
    k_iota_row = lax.broadcasted_iota(jnp.int32, (S, k), 1)
    s_iota_sk = lax.broadcasted_iota(jnp.int32, (S, k), 0)
    w_col = w_tok.T

    def iter_body(_, ctr):
        d = d_ref[...]
        oh_ctr = (s_iota_col == ctr).astype(jnp.float32)
        i2c = lax.dot(d, oh_ctr, preferred_element_type=jnp.float32)
        mn = jnp.min(i2c, axis=1, keepdims=True)
        assign = jnp.min(
            jnp.where(i2c == mn, k_iota_row, k), axis=1, keepdims=True
        )
        oh_a = (assign == k_iota_row).astype(jnp.float32)
        cost = lax.dot(
            d, oh_a * w_col, preferred_element_type=jnp.float32
        )
        cost = jnp.where(oh_a > 0, cost, jnp.inf)
        mnc = jnp.min(cost, axis=0, keepdims=True)
        ctr = jnp.min(
            jnp.where(cost == mnc, s_iota_sk, S), axis=0, keepdims=True
        )
        return ctr

    ctr = lax.fori_loop(0, _ITERS, iter_body, ctr)

    oh_ctr = (s_iota_col == ctr).astype(jnp.float32)
    out_ref[0] = lax.dot(xb, oh_ctr, preferred_element_type=jnp.float32)


@jax.jit
def kernel(x, w):
    B, F, S = x.shape
    k = _K
    if k >= S:
        return x
    return pl.pallas_call(
        _kmedoids_body,
        grid=(B,),
        in_specs=[
            pl.BlockSpec((1, F, S), lambda b: (b, 0, 0)),
            pl.BlockSpec((1, S, S), lambda b: (b, 0, 0)),
        ],
        out_specs=pl.BlockSpec((1, F, k), lambda b: (b, 0, 0)),
        out_shape=jax.ShapeDtypeStruct((B, F, k), x.dtype),
        scratch_shapes=[
            pltpu.VMEM((F // 8, S, 8), jnp.float32),
            pltpu.VMEM((S, S), jnp.float32),
        ],
    )(x, w)


# speedup vs baseline: 1.6934x; 1.6934x over previous
import functools

import jax
import jax.numpy as jnp
from jax import lax
from jax.experimental import pallas as pl
from jax.experimental.pallas import tpu as pltpu

_K = 16
_ITERS = 3


def _kmedoids_body(x_ref, w_ref, out_ref, xt3_ref, d_ref):
    xb = x_ref[0]
    wb = w_ref[0]
    F, S = xb.shape
    k = _K
    G = F // 8
    CH = 64

    w_tok = jnp.mean(wb, axis=0, keepdims=True)

    for g in range(G):
        xt3_ref[g] = xb[8 * g : 8 * g + 8, :].T

    H = S // 2

    def ci_top_body(ci, _):
        acc = jnp.zeros((CH, S), jnp.float32)
        for g in range(G):
            xg = x_ref[0, 8 * g : 8 * g + 8, :]
            xtc = xt3_ref[g, pl.ds(ci * CH, CH), :]
            for u in range(8):
                col = xtc[:, u : u + 1]
                row = xg[u : u + 1, :]
                acc = acc + jnp.minimum(col, row)
        d_ref[pl.ds(ci * CH, CH), :] = acc
        return 0

    lax.fori_loop(0, H // CH, ci_top_body, 0)

    def ci_bot_body(ci, _):
        acc = jnp.zeros((CH, H), jnp.float32)
        for g in range(G):
            xg = x_ref[0, 8 * g : 8 * g + 8, H:]
            xtc = xt3_ref[g, pl.ds(H + ci * CH, CH), :]
            for u in range(8):
                col = xtc[:, u : u + 1]
                row = xg[u : u + 1, :]
                acc = acc + jnp.minimum(col, row)
        d_ref[pl.ds(H + ci * CH, CH), H:] = acc
        return 0

    lax.fori_loop(0, H // CH, ci_bot_body, 0)

    d_ref[H:, :H] = d_ref[:H, H:].T

    s_row = jnp.sum(xb, axis=0, keepdims=True)
    s_col = s_row.T
    d_ref[...] = s_col + s_row - 2.0 * d_ref[...]

    k_iota = lax.broadcasted_iota(jnp.int32, (1, k), 1)
    w_row = w_tok
    w_colv = w_tok.T
    lane_iota2 = lax.broadcasted_iota(jnp.int32, (S, S), 1)
    sub_iota2 = lax.broadcasted_iota(jnp.int32, (S, S), 0)
    beats = jnp.logical_or(
        w_row > w_colv,
        jnp.logical_and(w_row == w_colv, lane_iota2 < sub_iota2),
    ).astype(jnp.float32)
    rank_col = lax.dot(
        beats, jnp.ones((S, 1), jnp.float32), preferred_element_type=jnp.float32
    )
    onehot_rank = (rank_col == k_iota.astype(jnp.float32)).astype(
        jnp.float32
    )
    iota_row_s = lax.broadcasted_iota(jnp.int32, (1, S), 1).astype(jnp.float32)
    ctr = lax.dot(
        iota_row_s, onehot_rank, preferred_element_type=jnp.float32
    ).astype(jnp.int32)

    s_iota_col = lax.broadcasted_iota(jnp.int32, (S, 1), 0)
    k_iota_row = lax.broadcasted_iota(jnp.int32, (S, k), 1)
    s_iota_sk = lax.broadcasted_iota(jnp.int32, (S, k), 0)
    w_col = w_tok.T

    def iter_body(_, ctr):
        d = d_ref[...]
        oh_ctr = (s_iota_col == ctr).astype(jnp.float32)
        i2c = lax.dot(
            d,
            oh_ctr,
            precision=lax.Precision.HIGHEST,
            preferred_element_type=jnp.float32,
        )
        mn = jnp.min(i2c, axis=1, keepdims=True)
        assign = jnp.min(
            jnp.where(i2c == mn, k_iota_row, k), axis=1, keepdims=True
        )
        oh_a = (assign == k_iota_row).astype(jnp.float32)
        cost = lax.dot(
            d, oh_a * w_col, preferred_element_type=jnp.float32
        )
        cost = jnp.where(oh_a > 0, cost, jnp.inf)
        mnc = jnp.min(cost, axis=0, keepdims=True)
        ctr = jnp.min(
            jnp.where(cost == mnc, s_iota_sk, S), axis=0, keepdims=True
        )
        return ctr

    ctr = lax.fori_loop(0, _ITERS, iter_body, ctr)

    oh_ctr = (s_iota_col == ctr).astype(jnp.float32)
    out_ref[0] = lax.dot(
        xb,
        oh_ctr,
        precision=lax.Precision.HIGHEST,
        preferred_element_type=jnp.float32,
    )


@jax.jit
def kernel(x, w):
    B, F, S = x.shape
    k = _K
    if k >= S:
        return x
    return pl.pallas_call(
        _kmedoids_body,
        grid=(B,),
        in_specs=[
            pl.BlockSpec((1, F, S), lambda b: (b, 0, 0)),
            pl.BlockSpec((1, S, S), lambda b: (b, 0, 0)),
        ],
        out_specs=pl.BlockSpec((1, F, k), lambda b: (b, 0, 0)),
        out_shape=jax.ShapeDtypeStruct((B, F, k), x.dtype),
        scratch_shapes=[
            pltpu.VMEM((F // 8, S, 8), jnp.float32),
            pltpu.VMEM((S, S), jnp.float32),
        ],
    )(x, w)


# speedup vs baseline: 1.7232x; 1.0176x over previous
import functools

import jax
import jax.numpy as jnp
from jax import lax
from jax.experimental import pallas as pl
from jax.experimental.pallas import tpu as pltpu

_K = 16
_ITERS = 3


def _kmedoids_body(x_ref, w_ref, out_ref, xt3_ref, d_ref):
    xb = x_ref[0]
    wb = w_ref[0]
    F, S = xb.shape
    k = _K
    G = F // 8
    CH = 64

    w_tok = jnp.mean(wb, axis=0, keepdims=True)

    for g in range(G):
        xt3_ref[g] = xb[8 * g : 8 * g + 8, :].T

    H = S // 2

    def ci_top_body(ci, _):
        acc0 = jnp.zeros((CH, S), jnp.float32)
        acc1 = jnp.zeros((CH, S), jnp.float32)
        for g in range(G):
            xg = x_ref[0, 8 * g : 8 * g + 8, :]
            xtc = xt3_ref[g, pl.ds(ci * CH, CH), :]
            for u in range(0, 8, 2):
                acc0 = acc0 + jnp.minimum(xtc[:, u : u + 1], xg[u : u + 1, :])
                acc1 = acc1 + jnp.minimum(
                    xtc[:, u + 1 : u + 2], xg[u + 1 : u + 2, :]
                )
        d_ref[pl.ds(ci * CH, CH), :] = acc0 + acc1
        return 0

    lax.fori_loop(0, H // CH, ci_top_body, 0)

    def ci_bot_body(ci, _):
        acc0 = jnp.zeros((CH, H), jnp.float32)
        acc1 = jnp.zeros((CH, H), jnp.float32)
        for g in range(G):
            xg = x_ref[0, 8 * g : 8 * g + 8, H:]
            xtc = xt3_ref[g, pl.ds(H + ci * CH, CH), :]
            for u in range(0, 8, 2):
                acc0 = acc0 + jnp.minimum(xtc[:, u : u + 1], xg[u : u + 1, :])
                acc1 = acc1 + jnp.minimum(
                    xtc[:, u + 1 : u + 2], xg[u + 1 : u + 2, :]
                )
        d_ref[pl.ds(H + ci * CH, CH), H:] = acc0 + acc1
        return 0

    lax.fori_loop(0, H // CH, ci_bot_body, 0)

    d_ref[H:, :H] = d_ref[:H, H:].T

    s_row = jnp.sum(xb, axis=0, keepdims=True)
    s_col = s_row.T
    d_ref[...] = s_col + s_row - 2.0 * d_ref[...]

    k_iota = lax.broadcasted_iota(jnp.int32, (1, k), 1)
    w_row = w_tok
    w_colv = w_tok.T
    lane_iota2 = lax.broadcasted_iota(jnp.int32, (S, S), 1)
    sub_iota2 = lax.broadcasted_iota(jnp.int32, (S, S), 0)
    beats = jnp.logical_or(
        w_row > w_colv,
        jnp.logical_and(w_row == w_colv, lane_iota2 < sub_iota2),
    ).astype(jnp.float32)
    rank_col = lax.dot(
        beats, jnp.ones((S, 1), jnp.float32), preferred_element_type=jnp.float32
    )
    onehot_rank = (rank_col == k_iota.astype(jnp.float32)).astype(
        jnp.float32
    )
    iota_row_s = lax.broadcasted_iota(jnp.int32, (1, S), 1).astype(jnp.float32)
    ctr = lax.dot(
        iota_row_s, onehot_rank, preferred_element_type=jnp.float32
    ).astype(jnp.int32)

    s_iota_col = lax.broadcasted_iota(jnp.int32, (S, 1), 0)
    k_iota_row = lax.broadcasted_iota(jnp.int32, (S, k), 1)
    s_iota_sk = lax.broadcasted_iota(jnp.int32, (S, k), 0)
    w_col = w_tok.T

    def iter_body(_, ctr):
        d = d_ref[...]
        oh_ctr = (s_iota_col == ctr).astype(jnp.float32)
        i2c = lax.dot(
            d,
            oh_ctr,
            precision=lax.Precision.HIGHEST,
            preferred_element_type=jnp.float32,
        )
        mn = jnp.min(i2c, axis=1, keepdims=True)
        assign = jnp.min(
            jnp.where(i2c == mn, k_iota_row, k), axis=1, keepdims=True
        )
        oh_a = (assign == k_iota_row).astype(jnp.float32)
        cost = lax.dot(
            d, oh_a * w_col, preferred_element_type=jnp.float32
        )
        cost = jnp.where(oh_a > 0, cost, jnp.inf)
        mnc = jnp.min(cost, axis=0, keepdims=True)
        ctr = jnp.min(
            jnp.where(cost == mnc, s_iota_sk, S), axis=0, keepdims=True
        )
        return ctr

    ctr = lax.fori_loop(0, _ITERS, iter_body, ctr)

    oh_ctr = (s_iota_col == ctr).astype(jnp.float32)
    out_ref[0] = lax.dot(
        xb,
        oh_ctr,
        precision=lax.Precision.HIGHEST,
        preferred_element_type=jnp.float32,
    )


@jax.jit
def kernel(x, w):
    B, F, S = x.shape
    k = _K
    if k >= S:
        return x
    return pl.pallas_call(
        _kmedoids_body,
        grid=(B,),
        in_specs=[
            pl.BlockSpec((1, F, S), lambda b: (b, 0, 0)),
            pl.BlockSpec((1, S, S), lambda b: (b, 0, 0)),
        ],
        out_specs=pl.BlockSpec((1, F, k), lambda b: (b, 0, 0)),
        out_shape=jax.ShapeDtypeStruct((B, F, k), x.dtype),
        scratch_shapes=[
            pltpu.VMEM((F // 8, S, 8), jnp.float32),
            pltpu.VMEM((S, S), jnp.float32),
        ],
    )(x, w)
